# Initial kernel scaffold; baseline (speedup 1.0000x reference)
import jax
import jax.numpy as jnp
from jax.experimental import pallas as pl


def kernel(features_0, features_1, features_2, user_emb, loc_emb, add_emb1, lin_W, lin_b, proj_W, proj_b):
    raise NotImplementedError("write your pallas kernel here")


# speedup vs baseline: 1.0260x; 1.0260x over previous
import math

import jax
import jax.numpy as jnp
from jax import lax
from jax.experimental import pallas as pl
from jax.experimental.pallas import tpu as pltpu
from jax.experimental.pallas import tpu_sc as plsc

NC, NS = 2, 16
NW = NC * NS
B, L = 4096, 50
SEQ = L + 1
E0, E1 = 64, 16
D = 128
CH = 128

B_PER_W = B // NW
LOC_PER_W = B_PER_W * L
A1_PER_W = B_PER_W * SEQ
N_LOC_CH = LOC_PER_W // CH
N_A1_CH = A1_PER_W // CH


def _sc_gather_body(loc_idx, user_idx, a1_idx, loc_emb, user_emb, add_emb1,
                    loc_out, user_out, a1_out,
                    lidx_v, uidx_v, aidx_v, lrow_v, urow_v, arow_v, sem):
  wid = lax.axis_index("s") * NC + lax.axis_index("c")

  pltpu.sync_copy(loc_idx.at[wid], lidx_v)
  pltpu.sync_copy(user_idx.at[wid], uidx_v)
  pltpu.sync_copy(a1_idx.at[wid], aidx_v)

  pltpu.async_copy(user_emb.at[uidx_v], urow_v, sem).wait()
  pltpu.sync_copy(urow_v, user_out.at[wid])

  def loc_step(j, carry):
    pltpu.async_copy(loc_emb.at[lidx_v.at[j]], lrow_v, sem).wait()
    pltpu.sync_copy(lrow_v, loc_out.at[wid, j])
    return carry

  lax.fori_loop(0, N_LOC_CH, loc_step, 0)

  def a1_step(j, carry):
    pltpu.async_copy(add_emb1.at[aidx_v.at[j]], arow_v, sem).wait()
    pltpu.sync_copy(arow_v, a1_out.at[wid, j])
    return carry

  lax.fori_loop(0, N_A1_CH, a1_step, 0)


def _sc_gather(loc_idx, user_idx, a1_idx, loc_emb, user_emb, add_emb1):
  mesh = plsc.VectorSubcoreMesh(core_axis_name="c", subcore_axis_name="s",
                                num_cores=NC, num_subcores=NS)
  fn = pl.kernel(
      _sc_gather_body,
      out_type=(
          jax.ShapeDtypeStruct((NW, N_LOC_CH, CH, E0), jnp.float32),
          jax.ShapeDtypeStruct((NW, CH, E0), jnp.float32),
          jax.ShapeDtypeStruct((NW, N_A1_CH, CH, E1), jnp.float32),
      ),
      mesh=mesh,
      scratch_types=[
          pltpu.VMEM((N_LOC_CH, CH), jnp.int32),
          pltpu.VMEM((CH,), jnp.int32),
          pltpu.VMEM((N_A1_CH, CH), jnp.int32),
          pltpu.VMEM((CH, E0), jnp.float32),
          pltpu.VMEM((CH, E0), jnp.float32),
          pltpu.VMEM((CH, E1), jnp.float32),
          pltpu.SemaphoreType.DMA,
      ],
      compiler_params=pltpu.CompilerParams(use_tc_tiling_on_sc=False),
  )
  return fn(loc_idx, user_idx, a1_idx, loc_emb, user_emb, add_emb1)


BB = 128


def _tc_fuse_body(l_ref, u_ref, a_ref, f_ref, w0_ref, w1_ref, v_ref, c_ref,
                  o_ref):
  w0 = w0_ref[...]
  xl = l_ref[...].reshape(BB * L, E0)
  out_l = jnp.dot(xl, w0, preferred_element_type=jnp.float32)
  out_l = out_l.reshape(BB, L, D)
  out_u = jnp.dot(u_ref[...], w0, preferred_element_type=jnp.float32)
  xa = a_ref[...].reshape(BB * SEQ, E1)
  out_a = jnp.dot(xa, w1_ref[...], preferred_element_type=jnp.float32)
  out_a = out_a.reshape(BB, SEQ, D)
  e0_part = jnp.concatenate([out_u[:, None, :], out_l], axis=1)
  o_ref[...] = (e0_part + out_a
                + f_ref[...][..., None] * v_ref[...][None]
                + c_ref[...][None])


def _tc_fuse(loc_g, user_g, a1_g, f2, w0, w1, vrow, crow):
  grid = (B // BB,)
  return pl.pallas_call(
      _tc_fuse_body,
      grid=grid,
      in_specs=[
          pl.BlockSpec((BB, L, E0), lambda i: (i, 0, 0)),
          pl.BlockSpec((BB, E0), lambda i: (i, 0)),
          pl.BlockSpec((BB, SEQ, E1), lambda i: (i, 0, 0)),
          pl.BlockSpec((BB, SEQ), lambda i: (i, 0)),
          pl.BlockSpec((E0, D), lambda i: (0, 0)),
          pl.BlockSpec((E1, D), lambda i: (0, 0)),
          pl.BlockSpec((1, D), lambda i: (0, 0)),
          pl.BlockSpec((1, D), lambda i: (0, 0)),
      ],
      out_specs=pl.BlockSpec((BB, SEQ, D), lambda i: (i, 0, 0)),
      out_shape=jax.ShapeDtypeStruct((B, SEQ, D), jnp.float32),
  )(loc_g, user_g, a1_g, f2, w0, w1, vrow, crow)


def kernel(features_0, features_1, features_2, user_emb, loc_emb, add_emb1,
           lin_W, lin_b, proj_W, proj_b):
  s = math.sqrt(D)
  f0 = features_0.astype(jnp.int32)
  f1 = features_1.astype(jnp.int32)

  user_idx = f0[:, 0].reshape(NW, CH)
  loc_idx = f0[:, 1:].reshape(NW, N_LOC_CH, CH)
  a1_idx = f1.reshape(NW, N_A1_CH, CH)

  loc_g, user_g, a1_g = _sc_gather(loc_idx, user_idx, a1_idx,
                                   loc_emb, user_emb, add_emb1)
  loc_g = loc_g.reshape(B, L, E0)
  user_g = user_g.reshape(B, E0)
  a1_g = a1_g.reshape(B, SEQ, E1)

  w0 = proj_W[:, :E0].T * s
  w1 = proj_W[:, E0:E0 + E1].T * s
  w2t = proj_W[:, E0 + E1:].T
  vrow = (lin_W[:, 0] @ w2t).reshape(1, D) * s
  crow = ((lin_b @ w2t + proj_b) * s).reshape(1, D)

  f2 = features_2[..., 0]
  return _tc_fuse(loc_g, user_g, a1_g, f2, w0, w1, vrow, crow)


# speedup vs baseline: 1.3085x; 1.2754x over previous
import math

import jax
import jax.numpy as jnp
from jax import lax
from jax.experimental import pallas as pl
from jax.experimental.pallas import tpu as pltpu
from jax.experimental.pallas import tpu_sc as plsc

NC, NS = 2, 16
NW = NC * NS
B, L = 4096, 50
SEQ = L + 1
NTOK = B * SEQ
E0, E1 = 64, 16
D = 128
CH = 128

B_PER_W = B // NW
LOC_PER_W = B * L // NW
A1_PER_W = NTOK // NW
N_LOC_CH = LOC_PER_W // CH
N_A1_CH = A1_PER_W // CH


def _pipe_gather(table, idx_v, nch, bufs, gsems, ssems, out, base):
  pltpu.async_copy(table.at[idx_v.at[0]], bufs.at[0], gsems.at[0])

  def step(j, carry):
    p = lax.rem(j, 2)
    q = 1 - p

    @pl.when(j >= 1)
    def _wait_prev_store():
      pltpu.make_async_copy(
          bufs.at[q], out.at[pl.ds(base + (j - 1) * CH, CH)], ssems.at[q]
      ).wait()

    @pl.when(j + 1 < nch)
    def _fire_next_gather():
      pltpu.async_copy(table.at[idx_v.at[j + 1]], bufs.at[q], gsems.at[q])

    pltpu.make_async_copy(table.at[idx_v.at[j]], bufs.at[p], gsems.at[p]).wait()
    pltpu.async_copy(bufs.at[p], out.at[pl.ds(base + j * CH, CH)], ssems.at[p])
    return carry

  lax.fori_loop(0, nch, step, 0)
  last = (nch - 1) % 2
  pltpu.make_async_copy(
      bufs.at[last], out.at[pl.ds(base + (nch - 1) * CH, CH)], ssems.at[last]
  ).wait()


def _sc_gather_body(loc_idx, user_idx, a1_idx, loc_emb, user_emb, add_emb1,
                    e0_out, a1_out,
                    lidx_v, uidx_v, aidx_v, erow_v, arow_v,
                    gsems, ssems, usem):
  wid = lax.axis_index("s") * NC + lax.axis_index("c")

  pltpu.sync_copy(loc_idx.at[wid], lidx_v)
  pltpu.sync_copy(user_idx.at[wid], uidx_v)
  pltpu.sync_copy(a1_idx.at[wid], aidx_v)

  pltpu.async_copy(user_emb.at[uidx_v], erow_v.at[0], usem).wait()
  pltpu.sync_copy(erow_v.at[0], e0_out.at[pl.ds(wid * B_PER_W, B_PER_W)])

  _pipe_gather(loc_emb, lidx_v, N_LOC_CH, erow_v, gsems, ssems,
               e0_out, B + wid * LOC_PER_W)
  _pipe_gather(add_emb1, aidx_v, N_A1_CH, arow_v, gsems, ssems,
               a1_out, wid * A1_PER_W)


def _sc_gather(loc_idx, user_idx, a1_idx, loc_emb, user_emb, add_emb1):
  mesh = plsc.VectorSubcoreMesh(core_axis_name="c", subcore_axis_name="s",
                                num_cores=NC, num_subcores=NS)
  fn = pl.kernel(
      _sc_gather_body,
      out_type=(
          jax.ShapeDtypeStruct((NTOK, E0), jnp.float32),
          jax.ShapeDtypeStruct((NTOK, E1), jnp.float32),
      ),
      mesh=mesh,
      scratch_types=[
          pltpu.VMEM((N_LOC_CH, CH), jnp.int32),
          pltpu.VMEM((CH,), jnp.int32),
          pltpu.VMEM((N_A1_CH, CH), jnp.int32),
          pltpu.VMEM((2, CH, E0), jnp.float32),
          pltpu.VMEM((2, CH, E1), jnp.float32),
          pltpu.SemaphoreType.DMA((2,)),
          pltpu.SemaphoreType.DMA((2,)),
          pltpu.SemaphoreType.DMA,
      ],
      compiler_params=pltpu.CompilerParams(use_tc_tiling_on_sc=False),
  )
  return fn(loc_idx, user_idx, a1_idx, loc_emb, user_emb, add_emb1)


def _tc_fuse_body(e0_ref, a1_ref, f2_ref, w0_ref, w1_ref, v_ref, c_ref,
                  o_ref):
  acc = jnp.dot(e0_ref[...], w0_ref[...], preferred_element_type=jnp.float32)
  acc += jnp.dot(a1_ref[...], w1_ref[...], preferred_element_type=jnp.float32)
  f2col = jnp.transpose(f2_ref[0])
  o_ref[...] = acc + f2col * v_ref[...] + c_ref[...]


def _tc_fuse(e0_t, a1_t, f2_t3, w0, w1, vrow, crow):
  return pl.pallas_call(
      _tc_fuse_body,
      grid=(SEQ,),
      in_specs=[
          pl.BlockSpec((B, E0), lambda i: (i, 0)),
          pl.BlockSpec((B, E1), lambda i: (i, 0)),
          pl.BlockSpec((1, 1, B), lambda i: (i, 0, 0)),
          pl.BlockSpec((E0, D), lambda i: (0, 0)),
          pl.BlockSpec((E1, D), lambda i: (0, 0)),
          pl.BlockSpec((1, D), lambda i: (0, 0)),
          pl.BlockSpec((1, D), lambda i: (0, 0)),
      ],
      out_specs=pl.BlockSpec((B, D), lambda i: (i, 0)),
      out_shape=jax.ShapeDtypeStruct((NTOK, D), jnp.float32),
  )(e0_t, a1_t, f2_t3, w0, w1, vrow, crow)


def kernel(features_0, features_1, features_2, user_emb, loc_emb, add_emb1,
           lin_W, lin_b, proj_W, proj_b):
  s = math.sqrt(D)
  f0_t = jnp.transpose(features_0.astype(jnp.int32))
  f1_t = jnp.transpose(features_1.astype(jnp.int32))

  user_idx = f0_t[0].reshape(NW, CH)
  loc_idx = f0_t[1:].reshape(NW, N_LOC_CH, CH)
  a1_idx = f1_t.reshape(NW, N_A1_CH, CH)

  e0_t, a1_t = _sc_gather(loc_idx, user_idx, a1_idx,
                          loc_emb, user_emb, add_emb1)

  w0 = proj_W[:, :E0].T * s
  w1 = proj_W[:, E0:E0 + E1].T * s
  w2t = proj_W[:, E0 + E1:].T
  vrow = (lin_W[:, 0] @ w2t).reshape(1, D) * s
  crow = ((lin_b @ w2t + proj_b) * s).reshape(1, D)

  f2_t3 = jnp.transpose(features_2[:, :, 0]).reshape(SEQ, 1, B)
  out_flat = _tc_fuse(e0_t, a1_t, f2_t3, w0, w1, vrow, crow)
  return jnp.transpose(out_flat.reshape(SEQ, B, D), (1, 0, 2))
